# Initial kernel scaffold; baseline (speedup 1.0000x reference)
import jax
import jax.numpy as jnp
from jax.experimental import pallas as pl


def kernel(verts, edges, verts_idx, w0_0, b0_0, w1_0, b1_0, w0_1, b0_1, w1_1, b1_1, fc1_w, fc1_b, fc2_w, fc2_b):
    raise NotImplementedError("write your pallas kernel here")


# speedup vs baseline: 2.0623x; 2.0623x over previous
import functools

import jax
import jax.numpy as jnp
from jax import lax
from jax.experimental import pallas as pl
from jax.experimental.pallas import tpu as pltpu
from jax.experimental.pallas import tpu_sc as plsc

_N, _D, _E, _B, _H, _C = 10000, 128, 320000, 8, 1024, 13
_NP = 10240
_BLK = 512
_NBLK = _NP // _BLK
_NC, _NS = 2, 16
_NW = _NC * _NS
_IDXW = 128
_R = 160
_RCH = 16
_MPAD = _NW * _R * _IDXW
_RPT = _NP // _NS


def _mm2_body(x_ref, w0_ref, b0_ref, w1_ref, b1_ref, o0_ref, o1_ref):
    x = x_ref[...]
    o0_ref[...] = jnp.dot(x, w0_ref[...], preferred_element_type=jnp.float32) + b0_ref[...]
    o1_ref[...] = jnp.dot(x, w1_ref[...], preferred_element_type=jnp.float32) + b1_ref[...]


def _combine_mm2_body(v0_ref, n0_ref, n1_ref, w0_ref, b0_ref, w1_ref, b1_ref,
                      o0_ref, o1_ref):
    h = jnp.maximum(v0_ref[...] + n0_ref[...] + n1_ref[...], 0.0)
    o0_ref[...] = jnp.dot(h, w0_ref[...], preferred_element_type=jnp.float32) + b0_ref[...]
    o1_ref[...] = jnp.dot(h, w1_ref[...], preferred_element_type=jnp.float32) + b1_ref[...]


_w_spec = pl.BlockSpec((_D, _D), lambda i: (0, 0))
_b_spec = pl.BlockSpec((1, _D), lambda i: (0, 0))
_row_spec = pl.BlockSpec((_BLK, _D), lambda i: (i, 0))
_nbr0_spec = pl.BlockSpec((_BLK, _D), lambda i: (i, 0))
_nbr1_spec = pl.BlockSpec((_BLK, _D), lambda i: (_NBLK + i, 0))

_mm2 = pl.pallas_call(
    _mm2_body,
    grid=(_NBLK,),
    in_specs=[_row_spec, _w_spec, _b_spec, _w_spec, _b_spec],
    out_specs=[_row_spec, _row_spec],
    out_shape=[jax.ShapeDtypeStruct((_NP, _D), jnp.float32)] * 2,
)

_combine_mm2 = pl.pallas_call(
    _combine_mm2_body,
    grid=(_NBLK,),
    in_specs=[_row_spec, _nbr0_spec, _nbr1_spec, _w_spec, _b_spec, _w_spec, _b_spec],
    out_specs=[_row_spec, _row_spec],
    out_shape=[jax.ShapeDtypeStruct((_NP, _D), jnp.float32)] * 2,
)


def _sc_scatter_body(vw1_hbm, gidx_hbm, tidx_hbm, zeros_hbm, out_hbm,
                     gidx_v, tidx_v, rows_v, acc_sh):
    c = lax.axis_index("c")
    s = lax.axis_index("s")
    wid = s * _NC + c
    pltpu.sync_copy(zeros_hbm.at[pl.ds(s * _RPT, _RPT)],
                    acc_sh.at[pl.ds(s * _RPT, _RPT)])
    plsc.subcore_barrier()

    def outer(j, carry):
        pltpu.sync_copy(gidx_hbm.at[wid, pl.ds(j * _RCH, _RCH)], gidx_v)
        pltpu.sync_copy(tidx_hbm.at[wid, pl.ds(j * _RCH, _RCH)], tidx_v)

        def inner(b, c2):
            pltpu.sync_copy(vw1_hbm.at[gidx_v.at[b]], rows_v)
            pltpu.sync_copy(rows_v, acc_sh.at[tidx_v.at[b]], add=True)
            return c2

        return lax.fori_loop(0, _RCH, inner, carry)

    lax.fori_loop(0, _R // _RCH, outer, 0)
    plsc.subcore_barrier()
    base = c * _NP + s * _RPT
    pltpu.sync_copy(acc_sh.at[pl.ds(s * _RPT, _RPT)],
                    out_hbm.at[pl.ds(base, _RPT)])


@functools.cache
def _get_sc_scatter():
    return pl.kernel(
        _sc_scatter_body,
        out_type=jax.ShapeDtypeStruct((_NC * _NP, _D), jnp.float32),
        mesh=plsc.VectorSubcoreMesh(core_axis_name="c", subcore_axis_name="s"),
        scratch_types=[
            pltpu.VMEM((_RCH, _IDXW), jnp.int32),
            pltpu.VMEM((_RCH, _IDXW), jnp.int32),
            pltpu.VMEM((_IDXW, _D), jnp.float32),
            pltpu.VMEM_SHARED((_NP, _D), jnp.float32),
        ],
    )


def _final_body(v0_ref, n0_ref, n1_ref, seg_ref, fc1w_ref, fc1b_ref,
                fc2w_ref, fc2b_ref, out_ref, sums_ref, counts_ref):
    i = pl.program_id(0)

    @pl.when(i == 0)
    def _():
        sums_ref[...] = jnp.zeros_like(sums_ref)
        counts_ref[...] = jnp.zeros_like(counts_ref)

    h = jnp.maximum(v0_ref[...] + n0_ref[...] + n1_ref[...], 0.0)
    seg = seg_ref[0, 0, :]
    ids = lax.broadcasted_iota(jnp.int32, (_BLK, _B), 1)
    oh = (seg[:, None] == ids).astype(jnp.float32)
    sums_ref[...] += lax.dot_general(oh, h, (((0,), (0,)), ((), ())),
                                     preferred_element_type=jnp.float32)
    counts_ref[...] += lax.dot_general(oh, jnp.ones_like(h),
                                       (((0,), (0,)), ((), ())),
                                       preferred_element_type=jnp.float32)

    @pl.when(i == _NBLK - 1)
    def _():
        max_count = jnp.max(counts_ref[...])
        pooled = sums_ref[...] / max_count
        t = jnp.maximum(
            jnp.dot(pooled, fc1w_ref[...], preferred_element_type=jnp.float32)
            + fc1b_ref[...], 0.0)
        out_ref[...] = jnp.dot(t, fc2w_ref[...],
                               preferred_element_type=jnp.float32) + fc2b_ref[...]


_final = pl.pallas_call(
    _final_body,
    grid=(_NBLK,),
    in_specs=[
        _row_spec, _nbr0_spec, _nbr1_spec,
        pl.BlockSpec((1, 1, _BLK), lambda i: (i, 0, 0)),
        pl.BlockSpec((_D, _H), lambda i: (0, 0)),
        pl.BlockSpec((1, _H), lambda i: (0, 0)),
        pl.BlockSpec((_H, _D), lambda i: (0, 0)),
        pl.BlockSpec((1, _D), lambda i: (0, 0)),
    ],
    out_specs=pl.BlockSpec((_B, _D), lambda i: (0, 0)),
    out_shape=jax.ShapeDtypeStruct((_B, _D), jnp.float32),
    scratch_shapes=[pltpu.VMEM((_B, _D), jnp.float32),
                    pltpu.VMEM((_B, _D), jnp.float32)],
)


def kernel(verts, edges, verts_idx, w0_0, b0_0, w1_0, b1_0, w0_1, b0_1,
           w1_1, b1_1, fc1_w, fc1_b, fc2_w, fc2_b):
    f32 = jnp.float32
    verts_p = jnp.zeros((_NP, _D), f32).at[:_N, :].set(verts)

    src = edges[:, 0].astype(jnp.int32)
    dst = edges[:, 1].astype(jnp.int32)
    pad = jnp.full((_MPAD - 2 * _E,), _N, jnp.int32)
    gidx = jnp.concatenate([dst, src, pad]).reshape(_NW, _R, _IDXW)
    tidx = jnp.concatenate([src, dst, pad]).reshape(_NW, _R, _IDXW)
    zeros = jnp.zeros((_NP, _D), f32)

    seg = jnp.concatenate(
        [verts_idx.astype(jnp.int32), jnp.full((_NP - _N,), _B, jnp.int32)]
    ).reshape(_NBLK, 1, _BLK)

    fc2w_p = jnp.zeros((_H, _D), f32).at[:, :_C].set(fc2_w)
    fc2b_p = jnp.zeros((1, _D), f32).at[0, :_C].set(fc2_b)

    b2 = lambda b: b.reshape(1, _D)

    sc_scatter = _get_sc_scatter()
    vw0_a, vw1_a = _mm2(verts_p, w0_0, b2(b0_0), w1_0, b2(b1_0))
    nbr_a = sc_scatter(vw1_a, gidx, tidx, zeros)
    vw0_b, vw1_b = _combine_mm2(vw0_a, nbr_a, nbr_a, w0_1, b2(b0_1),
                                w1_1, b2(b1_1))
    nbr_b = sc_scatter(vw1_b, gidx, tidx, zeros)
    out = _final(vw0_b, nbr_b, nbr_b, seg, fc1_w, fc1_b.reshape(1, _H),
                 fc2w_p, fc2b_p)
    return out[:, :_C]


# speedup vs baseline: 3.1276x; 1.5165x over previous
import functools

import jax
import jax.numpy as jnp
from jax import lax
from jax.experimental import pallas as pl
from jax.experimental.pallas import tpu as pltpu
from jax.experimental.pallas import tpu_sc as plsc

_N, _D, _E, _B, _H, _C = 10000, 128, 320000, 8, 1024, 13
_NP = 10240
_BLK = 512
_NBLK = _NP // _BLK
_NC, _NS = 2, 16
_NW = _NC * _NS
_BW = 128
_SR = 20
_NBUF = 2
_SW = max(0, _NBUF - 2)
_NCHUNK = 256
_MPAD = _NCHUNK * _SR * _BW
_J0 = 10
_J1 = 16 - _J0
_RPT = _NP // _NS


def _mm2_body(x_ref, w0_ref, b0_ref, w1_ref, b1_ref, o0_ref, o1_ref):
    x = x_ref[...]
    o0_ref[...] = jnp.dot(x, w0_ref[...], preferred_element_type=jnp.float32) + b0_ref[...]
    o1_ref[...] = jnp.dot(x, w1_ref[...], preferred_element_type=jnp.float32) + b1_ref[...]


def _combine_mm2_body(v0_ref, n0_ref, n1_ref, w0_ref, b0_ref, w1_ref, b1_ref,
                      o0_ref, o1_ref):
    h = jnp.maximum(v0_ref[...] + n0_ref[...] + n1_ref[...], 0.0)
    o0_ref[...] = jnp.dot(h, w0_ref[...], preferred_element_type=jnp.float32) + b0_ref[...]
    o1_ref[...] = jnp.dot(h, w1_ref[...], preferred_element_type=jnp.float32) + b1_ref[...]


_w_spec = pl.BlockSpec((_D, _D), lambda i: (0, 0))
_b_spec = pl.BlockSpec((1, _D), lambda i: (0, 0))
_row_spec = pl.BlockSpec((_BLK, _D), lambda i: (i, 0))
_nbr0_spec = pl.BlockSpec((_BLK, _D), lambda i: (i, 0))
_nbr1_spec = pl.BlockSpec((_BLK, _D), lambda i: (_NBLK + i, 0))

_mm2 = pl.pallas_call(
    _mm2_body,
    grid=(_NBLK,),
    in_specs=[_row_spec, _w_spec, _b_spec, _w_spec, _b_spec],
    out_specs=[_row_spec, _row_spec],
    out_shape=[jax.ShapeDtypeStruct((_NP, _D), jnp.float32)] * 2,
)

_combine_mm2 = pl.pallas_call(
    _combine_mm2_body,
    grid=(_NBLK,),
    in_specs=[_row_spec, _nbr0_spec, _nbr1_spec, _w_spec, _b_spec, _w_spec, _b_spec],
    out_specs=[_row_spec, _row_spec],
    out_shape=[jax.ShapeDtypeStruct((_NP, _D), jnp.float32)] * 2,
)


def _sc_scatter_body(vw1_hbm, gidx_hbm, tidx_hbm, zeros_hbm, out_hbm,
                     gidx_v, tidx_v, rows_v, acc_sh, isem,
                     gsem0, gsem1, gsem2, gsem3,
                     ssem0, ssem1, ssem2, ssem3):
    c = lax.axis_index("c")
    s = lax.axis_index("s")
    cnt = jnp.where(c == 0, _J0, _J1)
    start = jnp.where(c == 0, s * _J0, _NS * _J0 + s * _J1)
    gsems = (gsem0, gsem1, gsem2, gsem3)
    ssems = (ssem0, ssem1, ssem2, ssem3)
    pltpu.sync_copy(zeros_hbm.at[pl.ds(s * _RPT, _RPT)],
                    acc_sh.at[pl.ds(s * _RPT, _RPT)])
    plsc.subcore_barrier()

    def outer(j, carry):
        pltpu.sync_copy(gidx_hbm.at[start + j], gidx_v)
        pltpu.sync_copy(tidx_hbm.at[start + j], tidx_v)

        gcp = [None] * _SR
        scp = [None] * _SR
        sdone = [False] * _SR
        for b in range(_NBUF):
            gcp[b] = pltpu.async_copy(vw1_hbm.at[gidx_v.at[b]],
                                      rows_v.at[b], gsems[b % _NBUF])
        for b in range(_SR):
            r = b % _NBUF
            gcp[b].wait()
            scp[b] = pltpu.async_copy(rows_v.at[r],
                                      acc_sh.at[tidx_v.at[b]],
                                      ssems[r], add=True)
            k = b + _NBUF - _SW
            if b >= _SW and k < _SR:
                scp[b - _SW].wait()
                sdone[b - _SW] = True
                gcp[k] = pltpu.async_copy(vw1_hbm.at[gidx_v.at[k]],
                                          rows_v.at[k % _NBUF],
                                          gsems[k % _NBUF])
        for b in range(_SR):
            if not sdone[b]:
                scp[b].wait()
        return carry

    lax.fori_loop(0, cnt, outer, 0)
    plsc.subcore_barrier()
    base = c * _NP + s * _RPT
    pltpu.sync_copy(acc_sh.at[pl.ds(s * _RPT, _RPT)],
                    out_hbm.at[pl.ds(base, _RPT)])


@functools.cache
def _get_sc_scatter():
    return pl.kernel(
        _sc_scatter_body,
        out_type=jax.ShapeDtypeStruct((_NC * _NP, _D), jnp.float32),
        mesh=plsc.VectorSubcoreMesh(core_axis_name="c", subcore_axis_name="s"),
        scratch_types=[
            pltpu.VMEM((_SR, _BW), jnp.int32),
            pltpu.VMEM((_SR, _BW), jnp.int32),
            pltpu.VMEM((_NBUF, _BW, _D), jnp.float32),
            pltpu.VMEM_SHARED((_NP, _D), jnp.float32),
        ] + [pltpu.SemaphoreType.DMA] * 9,
    )


def _final_body(v0_ref, n0_ref, n1_ref, seg_ref, fc1w_ref, fc1b_ref,
                fc2w_ref, fc2b_ref, out_ref, sums_ref, counts_ref):
    i = pl.program_id(0)

    @pl.when(i == 0)
    def _():
        sums_ref[...] = jnp.zeros_like(sums_ref)
        counts_ref[...] = jnp.zeros_like(counts_ref)

    h = jnp.maximum(v0_ref[...] + n0_ref[...] + n1_ref[...], 0.0)
    seg = seg_ref[0, 0, :]
    ids = lax.broadcasted_iota(jnp.int32, (_BLK, _B), 1)
    oh = (seg[:, None] == ids).astype(jnp.float32)
    sums_ref[...] += lax.dot_general(oh, h, (((0,), (0,)), ((), ())),
                                     preferred_element_type=jnp.float32)
    counts_ref[...] += lax.dot_general(oh, jnp.ones_like(h),
                                       (((0,), (0,)), ((), ())),
                                       preferred_element_type=jnp.float32)

    @pl.when(i == _NBLK - 1)
    def _():
        max_count = jnp.max(counts_ref[...])
        pooled = sums_ref[...] / max_count
        t = jnp.maximum(
            jnp.dot(pooled, fc1w_ref[...], preferred_element_type=jnp.float32)
            + fc1b_ref[...], 0.0)
        out_ref[...] = jnp.dot(t, fc2w_ref[...],
                               preferred_element_type=jnp.float32) + fc2b_ref[...]


_final = pl.pallas_call(
    _final_body,
    grid=(_NBLK,),
    in_specs=[
        _row_spec, _nbr0_spec, _nbr1_spec,
        pl.BlockSpec((1, 1, _BLK), lambda i: (i, 0, 0)),
        pl.BlockSpec((_D, _H), lambda i: (0, 0)),
        pl.BlockSpec((1, _H), lambda i: (0, 0)),
        pl.BlockSpec((_H, _D), lambda i: (0, 0)),
        pl.BlockSpec((1, _D), lambda i: (0, 0)),
    ],
    out_specs=pl.BlockSpec((_B, _D), lambda i: (0, 0)),
    out_shape=jax.ShapeDtypeStruct((_B, _D), jnp.float32),
    scratch_shapes=[pltpu.VMEM((_B, _D), jnp.float32),
                    pltpu.VMEM((_B, _D), jnp.float32)],
)


def kernel(verts, edges, verts_idx, w0_0, b0_0, w1_0, b1_0, w0_1, b0_1,
           w1_1, b1_1, fc1_w, fc1_b, fc2_w, fc2_b):
    f32 = jnp.float32
    verts_p = jnp.zeros((_NP, _D), f32).at[:_N, :].set(verts)

    src = edges[:, 0].astype(jnp.int32)
    dst = edges[:, 1].astype(jnp.int32)
    pad = jnp.full((_MPAD - 2 * _E,), _N, jnp.int32)
    gidx = jnp.concatenate([dst, src, pad]).reshape(_NCHUNK, _SR, _BW)
    tidx = jnp.concatenate([src, dst, pad]).reshape(_NCHUNK, _SR, _BW)
    zeros = jnp.zeros((_NP, _D), f32)

    seg = jnp.concatenate(
        [verts_idx.astype(jnp.int32), jnp.full((_NP - _N,), _B, jnp.int32)]
    ).reshape(_NBLK, 1, _BLK)

    fc2w_p = jnp.zeros((_H, _D), f32).at[:, :_C].set(fc2_w)
    fc2b_p = jnp.zeros((1, _D), f32).at[0, :_C].set(fc2_b)

    b2 = lambda b: b.reshape(1, _D)

    sc_scatter = _get_sc_scatter()
    vw0_a, vw1_a = _mm2(verts_p, w0_0, b2(b0_0), w1_0, b2(b1_0))
    nbr_a = sc_scatter(vw1_a, gidx, tidx, zeros)
    vw0_b, vw1_b = _combine_mm2(vw0_a, nbr_a, nbr_a, w0_1, b2(b0_1),
                                w1_1, b2(b1_1))
    nbr_b = sc_scatter(vw1_b, gidx, tidx, zeros)
    out = _final(vw0_b, nbr_b, nbr_b, seg, fc1_w, fc1_b.reshape(1, _H),
                 fc2w_p, fc2b_p)
    return out[:, :_C]


# speedup vs baseline: 3.1723x; 1.0143x over previous
import functools

import jax
import jax.numpy as jnp
from jax import lax
from jax.experimental import pallas as pl
from jax.experimental.pallas import tpu as pltpu
from jax.experimental.pallas import tpu_sc as plsc

_N, _D, _E, _B, _H, _C = 10000, 128, 320000, 8, 1024, 13
_NP = 10240
_BLK = 512
_NBLK = _NP // _BLK
_NC, _NS = 2, 16
_NW = _NC * _NS
_BW = 128
_SR = 20
_NBUF = 2
_SW = max(0, _NBUF - 2)
_NCHUNK = 256
_MPAD = _NCHUNK * _SR * _BW
_J0 = 12
_J1 = 16 - _J0
_RPT = _NP // _NS


def _mm2_body(x_ref, w0_ref, b0_ref, w1_ref, b1_ref, o0_ref, o1_ref):
    x = x_ref[...]
    o0_ref[...] = jnp.dot(x, w0_ref[...], preferred_element_type=jnp.float32) + b0_ref[...]
    o1_ref[...] = jnp.dot(x, w1_ref[...], preferred_element_type=jnp.float32) + b1_ref[...]


def _combine_mm2_body(v0_ref, n0_ref, n1_ref, w0_ref, b0_ref, w1_ref, b1_ref,
                      o0_ref, o1_ref):
    h = jnp.maximum(v0_ref[...] + n0_ref[...] + n1_ref[...], 0.0)
    o0_ref[...] = jnp.dot(h, w0_ref[...], preferred_element_type=jnp.float32) + b0_ref[...]
    o1_ref[...] = jnp.dot(h, w1_ref[...], preferred_element_type=jnp.float32) + b1_ref[...]


_w_spec = pl.BlockSpec((_D, _D), lambda i: (0, 0))
_b_spec = pl.BlockSpec((1, _D), lambda i: (0, 0))
_row_spec = pl.BlockSpec((_BLK, _D), lambda i: (i, 0))
_nbr0_spec = pl.BlockSpec((_BLK, _D), lambda i: (i, 0))
_nbr1_spec = pl.BlockSpec((_BLK, _D), lambda i: (_NBLK + i, 0))

_mm2 = pl.pallas_call(
    _mm2_body,
    grid=(_NBLK,),
    in_specs=[_row_spec, _w_spec, _b_spec, _w_spec, _b_spec],
    out_specs=[_row_spec, _row_spec],
    out_shape=[jax.ShapeDtypeStruct((_NP, _D), jnp.float32)] * 2,
)

_combine_mm2 = pl.pallas_call(
    _combine_mm2_body,
    grid=(_NBLK,),
    in_specs=[_row_spec, _nbr0_spec, _nbr1_spec, _w_spec, _b_spec, _w_spec, _b_spec],
    out_specs=[_row_spec, _row_spec],
    out_shape=[jax.ShapeDtypeStruct((_NP, _D), jnp.float32)] * 2,
)


def _sc_scatter_body(vw1_hbm, gidx_hbm, tidx_hbm, zeros_hbm, out_hbm,
                     gidx_v, tidx_v, rows_v, acc_sh, isem,
                     gsem0, gsem1, gsem2, gsem3,
                     ssem0, ssem1, ssem2, ssem3):
    c = lax.axis_index("c")
    s = lax.axis_index("s")
    cnt = jnp.where(c == 0, _J0, _J1)
    start = jnp.where(c == 0, s * _J0, _NS * _J0 + s * _J1)
    gsems = (gsem0, gsem1, gsem2, gsem3)
    ssems = (ssem0, ssem1, ssem2, ssem3)
    pltpu.sync_copy(zeros_hbm.at[pl.ds(s * _RPT, _RPT)],
                    acc_sh.at[pl.ds(s * _RPT, _RPT)])
    plsc.subcore_barrier()

    def outer(j, carry):
        pltpu.sync_copy(gidx_hbm.at[start + j], gidx_v)
        pltpu.sync_copy(tidx_hbm.at[start + j], tidx_v)

        gcp = [None] * _SR
        scp = [None] * _SR
        sdone = [False] * _SR
        for b in range(_NBUF):
            gcp[b] = pltpu.async_copy(vw1_hbm.at[gidx_v.at[b]],
                                      rows_v.at[b], gsems[b % _NBUF])
        for b in range(_SR):
            r = b % _NBUF
            gcp[b].wait()
            scp[b] = pltpu.async_copy(rows_v.at[r],
                                      acc_sh.at[tidx_v.at[b]],
                                      ssems[r], add=True)
            k = b + _NBUF - _SW
            if b >= _SW and k < _SR:
                scp[b - _SW].wait()
                sdone[b - _SW] = True
                gcp[k] = pltpu.async_copy(vw1_hbm.at[gidx_v.at[k]],
                                          rows_v.at[k % _NBUF],
                                          gsems[k % _NBUF])
        for b in range(_SR):
            if not sdone[b]:
                scp[b].wait()
        return carry

    lax.fori_loop(0, cnt, outer, 0)
    plsc.subcore_barrier()
    base = c * _NP + s * _RPT
    pltpu.sync_copy(acc_sh.at[pl.ds(s * _RPT, _RPT)],
                    out_hbm.at[pl.ds(base, _RPT)])


@functools.cache
def _get_sc_scatter():
    return pl.kernel(
        _sc_scatter_body,
        out_type=jax.ShapeDtypeStruct((_NC * _NP, _D), jnp.float32),
        mesh=plsc.VectorSubcoreMesh(core_axis_name="c", subcore_axis_name="s"),
        scratch_types=[
            pltpu.VMEM((_SR, _BW), jnp.int32),
            pltpu.VMEM((_SR, _BW), jnp.int32),
            pltpu.VMEM((_NBUF, _BW, _D), jnp.float32),
            pltpu.VMEM_SHARED((_NP, _D), jnp.float32),
        ] + [pltpu.SemaphoreType.DMA] * 9,
    )


def _final_body(v0_ref, n0_ref, n1_ref, seg_ref, fc1w_ref, fc1b_ref,
                fc2w_ref, fc2b_ref, out_ref, sums_ref, counts_ref):
    i = pl.program_id(0)

    @pl.when(i == 0)
    def _():
        sums_ref[...] = jnp.zeros_like(sums_ref)
        counts_ref[...] = jnp.zeros_like(counts_ref)

    h = jnp.maximum(v0_ref[...] + n0_ref[...] + n1_ref[...], 0.0)
    seg = seg_ref[0, 0, :]
    ids = lax.broadcasted_iota(jnp.int32, (_BLK, _B), 1)
    oh = (seg[:, None] == ids).astype(jnp.float32)
    sums_ref[...] += lax.dot_general(oh, h, (((0,), (0,)), ((), ())),
                                     preferred_element_type=jnp.float32)
    counts_ref[...] += lax.dot_general(oh, jnp.ones_like(h),
                                       (((0,), (0,)), ((), ())),
                                       preferred_element_type=jnp.float32)

    @pl.when(i == _NBLK - 1)
    def _():
        max_count = jnp.max(counts_ref[...])
        pooled = sums_ref[...] / max_count
        t = jnp.maximum(
            jnp.dot(pooled, fc1w_ref[...], preferred_element_type=jnp.float32)
            + fc1b_ref[...], 0.0)
        out_ref[...] = jnp.dot(t, fc2w_ref[...],
                               preferred_element_type=jnp.float32) + fc2b_ref[...]


_final = pl.pallas_call(
    _final_body,
    grid=(_NBLK,),
    in_specs=[
        _row_spec, _nbr0_spec, _nbr1_spec,
        pl.BlockSpec((1, 1, _BLK), lambda i: (i, 0, 0)),
        pl.BlockSpec((_D, _H), lambda i: (0, 0)),
        pl.BlockSpec((1, _H), lambda i: (0, 0)),
        pl.BlockSpec((_H, _D), lambda i: (0, 0)),
        pl.BlockSpec((1, _D), lambda i: (0, 0)),
    ],
    out_specs=pl.BlockSpec((_B, _D), lambda i: (0, 0)),
    out_shape=jax.ShapeDtypeStruct((_B, _D), jnp.float32),
    scratch_shapes=[pltpu.VMEM((_B, _D), jnp.float32),
                    pltpu.VMEM((_B, _D), jnp.float32)],
)


def kernel(verts, edges, verts_idx, w0_0, b0_0, w1_0, b1_0, w0_1, b0_1,
           w1_1, b1_1, fc1_w, fc1_b, fc2_w, fc2_b):
    f32 = jnp.float32
    verts_p = jnp.zeros((_NP, _D), f32).at[:_N, :].set(verts)

    src = edges[:, 0].astype(jnp.int32)
    dst = edges[:, 1].astype(jnp.int32)
    pad = jnp.full((_MPAD - 2 * _E,), _N, jnp.int32)
    gidx = jnp.concatenate([dst, src, pad]).reshape(_NCHUNK, _SR, _BW)
    tidx = jnp.concatenate([src, dst, pad]).reshape(_NCHUNK, _SR, _BW)
    zeros = jnp.zeros((_NP, _D), f32)

    seg = jnp.concatenate(
        [verts_idx.astype(jnp.int32), jnp.full((_NP - _N,), _B, jnp.int32)]
    ).reshape(_NBLK, 1, _BLK)

    fc2w_p = jnp.zeros((_H, _D), f32).at[:, :_C].set(fc2_w)
    fc2b_p = jnp.zeros((1, _D), f32).at[0, :_C].set(fc2_b)

    b2 = lambda b: b.reshape(1, _D)

    sc_scatter = _get_sc_scatter()
    vw0_a, vw1_a = _mm2(verts_p, w0_0, b2(b0_0), w1_0, b2(b1_0))
    nbr_a = sc_scatter(vw1_a, gidx, tidx, zeros)
    vw0_b, vw1_b = _combine_mm2(vw0_a, nbr_a, nbr_a, w0_1, b2(b0_1),
                                w1_1, b2(b1_1))
    nbr_b = sc_scatter(vw1_b, gidx, tidx, zeros)
    out = _final(vw0_b, nbr_b, nbr_b, seg, fc1_w, fc1_b.reshape(1, _H),
                 fc2w_p, fc2b_p)
    return out[:, :_C]
